# baseline (speedup 1.0000x reference)
import functools

import jax
import jax.numpy as jnp
from jax import lax
from jax.experimental import pallas as pl
from jax.experimental.pallas import tpu as pltpu
from jax.experimental.pallas import tpu_sc as plsc

N_VOX = 50000
N_PTS = 100000
D = 128
N_CAP = 128
N_PAIRS = 200000

BLK = 512
VOX_PAD = 50176
NW = 32
CHUNK = 128
CH_PER_TILE = 50
PER_TILE = CH_PER_TILE * CHUNK
PAIRS_PAD = NW * PER_TILE
SEG_BLK = 3200


def _table_body(s_ref, x_ref, cap_ref, out_ref):
    x = x_ref[...]
    cap = cap_ref[...]
    capn = cap / (jnp.sqrt(jnp.sum(cap * cap, axis=1, keepdims=True)) + 1e-12)
    nx = x / (jnp.sqrt(jnp.sum(x * x, axis=1, keepdims=True)) + 1e-12)
    s = jnp.exp(s_ref[0, 0])
    sc = lax.dot_general(nx, capn, (((1,), (1,)), ((), ())),
                         preferred_element_type=jnp.float32) * s
    m = jnp.max(sc, axis=1, keepdims=True)
    lse = m + jnp.log(jnp.sum(jnp.exp(sc - m), axis=1, keepdims=True))
    out_ref[...] = sc - lse


def _build_table(adapter_feats_pad, caption_embed, logit_scale):
    return pl.pallas_call(
        _table_body,
        grid=(VOX_PAD // BLK,),
        in_specs=[
            pl.BlockSpec((1, 1), lambda i: (0, 0)),
            pl.BlockSpec((BLK, D), lambda i: (i, 0)),
            pl.BlockSpec((N_CAP, D), lambda i: (0, 0)),
        ],
        out_specs=pl.BlockSpec((BLK, D), lambda i: (i, 0)),
        out_shape=jax.ShapeDtypeStruct((VOX_PAD, D), jnp.float32),
    )(logit_scale.reshape(1, 1), adapter_feats_pad, caption_embed)


def _sc_body(table, v2p, c2p, seg, zeros, out, c2p_v, seg_v, ub0, ub1,
             rows, acc, su0, su1):
    cid = lax.axis_index("c")
    sid = lax.axis_index("s")
    wid = cid * 16 + sid

    @pl.when(sid == 0)
    def _init():
        pltpu.sync_copy(zeros, acc)

    pltpu.sync_copy(c2p.at[wid], c2p_v)
    pltpu.sync_copy(seg.at[wid], seg_v)
    plsc.subcore_barrier()

    pltpu.async_copy(v2p.at[c2p_v.at[0]], ub0, su0)

    def body(j, carry):
        i0 = 2 * j
        i1 = 2 * j + 1
        pltpu.make_async_copy(v2p.at[c2p_v.at[0]], ub0, su0).wait()
        pltpu.async_copy(v2p.at[c2p_v.at[i1]], ub1, su1)
        pltpu.sync_copy(table.at[ub0], rows)
        pltpu.sync_copy(rows, acc.at[seg_v.at[i0]], add=True)
        pltpu.make_async_copy(v2p.at[c2p_v.at[0]], ub1, su1).wait()

        @pl.when(i0 + 2 < CH_PER_TILE)
        def _pre():
            pltpu.async_copy(v2p.at[c2p_v.at[i0 + 2]], ub0, su0)

        pltpu.sync_copy(table.at[ub1], rows)
        pltpu.sync_copy(rows, acc.at[seg_v.at[i1]], add=True)
        return carry

    lax.fori_loop(0, CH_PER_TILE // 2, body, 0)
    plsc.subcore_barrier()

    @pl.when(sid == 0)
    def _out():
        pltpu.sync_copy(acc, out.at[cid])


def _sc_segsum(table, v2p, c2p_pad, seg_pad, zeros):
    mesh = plsc.VectorSubcoreMesh(core_axis_name="c", subcore_axis_name="s")
    f = functools.partial(
        pl.kernel,
        mesh=mesh,
        out_type=jax.ShapeDtypeStruct((2, N_CAP + 1, D), jnp.float32),
        scratch_types=[
            pltpu.VMEM((CH_PER_TILE, CHUNK), jnp.int32),
            pltpu.VMEM((CH_PER_TILE, CHUNK), jnp.int32),
            pltpu.VMEM((CHUNK,), jnp.int32),
            pltpu.VMEM((CHUNK,), jnp.int32),
            pltpu.VMEM((CHUNK, D), jnp.float32),
            pltpu.VMEM_SHARED((N_CAP + 1, D), jnp.float32),
            pltpu.SemaphoreType.DMA,
            pltpu.SemaphoreType.DMA,
        ],
    )(_sc_body)
    return f(table, v2p, c2p_pad, seg_pad, zeros)


def _cnt_body(seg_ref, out_ref):
    i = pl.program_id(0)

    @pl.when(i == 0)
    def _init():
        out_ref[...] = jnp.zeros_like(out_ref)

    seg = seg_ref[...]
    caps = lax.broadcasted_iota(jnp.int32, (1, N_CAP), 1)
    onehot = (seg == caps).astype(jnp.float32)
    ones = jnp.ones((SEG_BLK, 1), jnp.float32)
    part = lax.dot_general(onehot, ones, (((0,), (0,)), ((), ())),
                           preferred_element_type=jnp.float32)
    out_ref[...] += part


def _counts(seg_pad_col):
    return pl.pallas_call(
        _cnt_body,
        grid=(PAIRS_PAD // SEG_BLK,),
        in_specs=[pl.BlockSpec((SEG_BLK, 1), lambda i: (i, 0))],
        out_specs=pl.BlockSpec((N_CAP, 1), lambda i: (0, 0)),
        out_shape=jax.ShapeDtypeStruct((N_CAP, 1), jnp.float32),
    )(seg_pad_col)


def _fin_body(g_ref, c_ref, out_ref, cnt_ref):
    g = g_ref[0] + g_ref[1]
    c = c_ref[...]
    out_ref[...] = g / jnp.maximum(c, 1.0)
    cnt_ref[...] = c


def _finalize(g2, cnt):
    return pl.pallas_call(
        _fin_body,
        out_shape=(
            jax.ShapeDtypeStruct((N_CAP, D), jnp.float32),
            jax.ShapeDtypeStruct((N_CAP, 1), jnp.float32),
        ),
    )(g2, cnt)


@jax.jit
def kernel(adapter_feats, v2p_map, caption_embed, caption2point_idx,
           segment_ids, logit_scale):
    feats_pad = jnp.pad(adapter_feats, ((0, VOX_PAD - N_VOX), (0, 0)))
    table = _build_table(feats_pad, caption_embed,
                         logit_scale.astype(jnp.float32))

    c2p_pad = jnp.pad(caption2point_idx.astype(jnp.int32),
                      (0, PAIRS_PAD - N_PAIRS))
    seg_pad = jnp.pad(segment_ids.astype(jnp.int32),
                      (0, PAIRS_PAD - N_PAIRS),
                      constant_values=N_CAP)
    zeros = jnp.zeros((N_CAP + 1, D), jnp.float32)

    acc2 = _sc_segsum(table, v2p_map.astype(jnp.int32),
                      c2p_pad.reshape(NW, CH_PER_TILE, CHUNK),
                      seg_pad.reshape(NW, CH_PER_TILE, CHUNK), zeros)

    cnt = _counts(seg_pad.reshape(PAIRS_PAD, 1))
    pooled, cnt2 = _finalize(acc2[:, :N_CAP, :], cnt)
    return pooled, cnt2.reshape(N_CAP)


# speedup vs baseline: 1.6731x; 1.6731x over previous
import functools

import jax
import jax.numpy as jnp
from jax import lax
from jax.experimental import pallas as pl
from jax.experimental.pallas import tpu as pltpu
from jax.experimental.pallas import tpu_sc as plsc

N_VOX = 50000
N_PTS = 100000
D = 128
N_CAP = 128
N_PAIRS = 200000

BLK = 512
VOX_PAD = 50176
NW = 32
CHUNK = 128
CH_PER_TILE = 49
PER_TILE = CH_PER_TILE * CHUNK
PAIRS_PAD = NW * PER_TILE
SEG_BLK = 3136


def _table_body(s_ref, x_ref, cap_ref, out_ref):
    x = x_ref[...]
    cap = cap_ref[...]
    capn = cap / (jnp.sqrt(jnp.sum(cap * cap, axis=1, keepdims=True)) + 1e-12)
    nx = x / (jnp.sqrt(jnp.sum(x * x, axis=1, keepdims=True)) + 1e-12)
    s = jnp.exp(s_ref[0, 0])
    sc = lax.dot_general(nx, capn, (((1,), (1,)), ((), ())),
                         preferred_element_type=jnp.float32) * s
    m = jnp.max(sc, axis=1, keepdims=True)
    lse = m + jnp.log(jnp.sum(jnp.exp(sc - m), axis=1, keepdims=True))
    out_ref[...] = sc - lse


def _build_table(adapter_feats_pad, caption_embed, logit_scale):
    return pl.pallas_call(
        _table_body,
        grid=(VOX_PAD // BLK,),
        in_specs=[
            pl.BlockSpec((1, 1), lambda i: (0, 0)),
            pl.BlockSpec((BLK, D), lambda i: (i, 0)),
            pl.BlockSpec((N_CAP, D), lambda i: (0, 0)),
        ],
        out_specs=pl.BlockSpec((BLK, D), lambda i: (i, 0)),
        out_shape=jax.ShapeDtypeStruct((VOX_PAD, D), jnp.float32),
    )(logit_scale.reshape(1, 1), adapter_feats_pad, caption_embed)


def _sc_body(table, v2p, c2p, seg, zeros, out, c2p_v, seg_v, ub, rows, acc):
    cid = lax.axis_index("c")
    sid = lax.axis_index("s")
    wid = cid * 16 + sid

    @pl.when(sid == 0)
    def _init():
        pltpu.sync_copy(zeros, acc)

    pltpu.sync_copy(c2p.at[wid], c2p_v)
    pltpu.sync_copy(seg.at[wid], seg_v)
    plsc.subcore_barrier()

    def body(i, carry):
        pltpu.sync_copy(v2p.at[c2p_v.at[i]], ub)
        pltpu.sync_copy(table.at[ub], rows)
        pltpu.sync_copy(rows, acc.at[seg_v.at[i]], add=True)
        return carry

    lax.fori_loop(0, CH_PER_TILE, body, 0)
    plsc.subcore_barrier()

    @pl.when(sid == 0)
    def _out():
        pltpu.sync_copy(acc, out.at[cid])


def _sc_segsum(table, v2p, c2p_pad, seg_pad, zeros):
    mesh = plsc.VectorSubcoreMesh(core_axis_name="c", subcore_axis_name="s")
    f = functools.partial(
        pl.kernel,
        mesh=mesh,
        out_type=jax.ShapeDtypeStruct((2, N_CAP + 1, D), jnp.float32),
        scratch_types=[
            pltpu.VMEM((CH_PER_TILE, CHUNK), jnp.int32),
            pltpu.VMEM((CH_PER_TILE, CHUNK), jnp.int32),
            pltpu.VMEM((CHUNK,), jnp.int32),
            pltpu.VMEM((CHUNK, D), jnp.float32),
            pltpu.VMEM_SHARED((N_CAP + 1, D), jnp.float32),
        ],
    )(_sc_body)
    return f(table, v2p, c2p_pad, seg_pad, zeros)


def _cnt_body(seg_ref, out_ref):
    i = pl.program_id(0)

    @pl.when(i == 0)
    def _init():
        out_ref[...] = jnp.zeros_like(out_ref)

    seg = seg_ref[...]
    caps = lax.broadcasted_iota(jnp.int32, (1, N_CAP), 1)
    onehot = (seg == caps).astype(jnp.float32)
    ones = jnp.ones((SEG_BLK, 1), jnp.float32)
    part = lax.dot_general(onehot, ones, (((0,), (0,)), ((), ())),
                           preferred_element_type=jnp.float32)
    out_ref[...] += part


def _counts(seg_pad_col):
    return pl.pallas_call(
        _cnt_body,
        grid=(PAIRS_PAD // SEG_BLK,),
        in_specs=[pl.BlockSpec((SEG_BLK, 1), lambda i: (i, 0))],
        out_specs=pl.BlockSpec((N_CAP, 1), lambda i: (0, 0)),
        out_shape=jax.ShapeDtypeStruct((N_CAP, 1), jnp.float32),
    )(seg_pad_col)


def _fin_body(g_ref, c_ref, out_ref, cnt_ref):
    g = g_ref[0] + g_ref[1]
    c = c_ref[...]
    out_ref[...] = g / jnp.maximum(c, 1.0)
    cnt_ref[...] = c


def _finalize(g2, cnt):
    return pl.pallas_call(
        _fin_body,
        out_shape=(
            jax.ShapeDtypeStruct((N_CAP, D), jnp.float32),
            jax.ShapeDtypeStruct((N_CAP, 1), jnp.float32),
        ),
    )(g2, cnt)


@jax.jit
def kernel(adapter_feats, v2p_map, caption_embed, caption2point_idx,
           segment_ids, logit_scale):
    feats_pad = jnp.pad(adapter_feats, ((0, VOX_PAD - N_VOX), (0, 0)))
    table = _build_table(feats_pad, caption_embed,
                         logit_scale.astype(jnp.float32))

    c2p_pad = jnp.pad(caption2point_idx.astype(jnp.int32),
                      (0, PAIRS_PAD - N_PAIRS))
    seg_pad = jnp.pad(segment_ids.astype(jnp.int32),
                      (0, PAIRS_PAD - N_PAIRS),
                      constant_values=N_CAP)
    zeros = jnp.zeros((N_CAP + 1, D), jnp.float32)

    acc2 = _sc_segsum(table, v2p_map.astype(jnp.int32),
                      c2p_pad.reshape(NW, CH_PER_TILE, CHUNK),
                      seg_pad.reshape(NW, CH_PER_TILE, CHUNK), zeros)

    cnt = _counts(seg_pad.reshape(PAIRS_PAD, 1))
    pooled, cnt2 = _finalize(acc2[:, :N_CAP, :], cnt)
    return pooled, cnt2.reshape(N_CAP)
